# Initial kernel scaffold; baseline (speedup 1.0000x reference)
import jax
import jax.numpy as jnp
from jax.experimental import pallas as pl


def kernel(xyz, neighbors, data_idxes, local_axises, cls_label, W0, b0, W02, b02, W1, b1, W12, b12, W2, b2, fc1_v, fc1_g, fc1_b, fc2_v, fc2_g, fc2_b, fc3_v, fc3_g, fc3_b):
    raise NotImplementedError("write your pallas kernel here")


# speedup vs baseline: 1.0609x; 1.0609x over previous
import functools
import jax
import jax.numpy as jnp
from jax.experimental import pallas as pl
from jax.experimental.pallas import tpu as pltpu

KNN = 17


def _gat(t, idx):
    B = t.shape[0]
    bidx = jnp.arange(B).reshape((B,) + (1,) * (idx.ndim - 1))
    return t[bidx, idx]


def _head_body(x_ref, v1_ref, s1_ref, b1_ref, v2_ref, s2_ref, b2_ref,
               v3_ref, s3_ref, b3_ref, o_ref):
    x = x_ref[...]
    h = jnp.dot(x, v1_ref[...].T, preferred_element_type=jnp.float32)
    h = jnp.maximum(h * s1_ref[...] + b1_ref[...], 0.0)
    h = jnp.dot(h, v2_ref[...].T, preferred_element_type=jnp.float32)
    h = jnp.maximum(h * s2_ref[...] + b2_ref[...], 0.0)
    h = jnp.dot(h, v3_ref[...].T, preferred_element_type=jnp.float32)
    h = h * s3_ref[...] + b3_ref[...]
    m = jnp.max(h, axis=-1, keepdims=True)
    z = h - m
    lse = jnp.log(jnp.sum(jnp.exp(z), axis=-1, keepdims=True))
    o_ref[...] = z - lse


def _head(x, fc1_v, fc1_g, fc1_b, fc2_v, fc2_g, fc2_b, fc3_v, fc3_g, fc3_b):
    R, C = x.shape
    BR = 2048
    s1 = (fc1_g / jnp.linalg.norm(fc1_v, axis=1))[None, :]
    s2 = (fc2_g / jnp.linalg.norm(fc2_v, axis=1))[None, :]
    s3 = (fc3_g / jnp.linalg.norm(fc3_v, axis=1))[None, :]
    full = lambda shape: pl.BlockSpec(shape, lambda i: (0, 0))
    return pl.pallas_call(
        _head_body,
        grid=(R // BR,),
        in_specs=[
            pl.BlockSpec((BR, C), lambda i: (i, 0)),
            full(fc1_v.shape), full((1, 512)), full((1, 512)),
            full(fc2_v.shape), full((1, 256)), full((1, 256)),
            full(fc3_v.shape), full((1, 40)), full((1, 40)),
        ],
        out_specs=pl.BlockSpec((BR, 40), lambda i: (i, 0)),
        out_shape=jax.ShapeDtypeStruct((R, 40), jnp.float32),
    )(x, fc1_v, s1, fc1_b[None, :], fc2_v, s2, fc2_b[None, :],
      fc3_v, s3, fc3_b[None, :])


def kernel(xyz, neighbors, data_idxes, local_axises, cls_label, W0, b0, W02,
           b02, W1, b1, W12, b12, W2, b2, fc1_v, fc1_g, fc1_b, fc2_v, fc2_g,
           fc2_b, fc3_v, fc3_g, fc3_b):
    B, N, _ = xyz.shape
    K = KNN
    nb = [neighbors[:, j * N:(j + 1) * N, 0:K] for j in range(4)]
    di = [data_idxes[:, j * N:(j + 1) * N] for j in range(4)]
    A = [local_axises[:, j * N:(j + 1) * N] for j in range(4)]

    e = [_gat(di[j][..., None], nb[j])[..., 0] for j in range(4)]

    c = [xyz]
    for j in range(3):
        c.append(_gat(c[j], di[j]))
    sdi = [di[0], di[0], di[1], di[2], di[3]]
    s = [xyz]
    for i in range(5):
        s.append(_gat(s[i], sdi[i]))

    lc = []
    for j in range(4):
        gx = _gat(c[j], e[j])
        cc = _gat(c[j], di[j])
        lc.append(jnp.einsum('bnkd,bndc->bnkc', gx - cc[:, :, None, :], A[j]))

    lc_cons = jnp.asarray(0.0, jnp.float32)
    for j in range(4):
        af = A[j].reshape(B, N, 9)
        ga = _gat(af, nb[j]).reshape(B * N, K, 3, 3)
        t = jnp.cos(jnp.asarray(15.0 * (j + 1) * 3.141592653 / 180.0,
                                jnp.float32))
        for col in (0, 1):
            v = ga[:, :, :, col]
            g = jnp.matmul(v, jnp.transpose(v, (0, 2, 1)))
            m = g < t
            cnt = jnp.sum(m)
            sm = jnp.sum(jnp.where(m, g, 0.0))
            lc_cons = lc_cons + jnp.where(
                cnt > 0, sm / jnp.maximum(cnt, 1).astype(jnp.float32), 0.0)

    Ws = [W0, W02, W1, W12, W2]
    bs = [b0, b02, b1, b12, b2]
    lidx = [0, 0, 1, 2, 3]
    p = None
    for i in range(5):
        W = Ws[i]
        Wl, Wg = W[0:3], W[3:6]
        T = jnp.einsum('bnd,dc->bnc', s[i], Wg)
        if p is not None:
            T = T + jnp.einsum('bnf,fc->bnc', p, W[6:])
        j = lidx[i]
        gT = _gat(T, e[j])
        L = jnp.einsum('bnkd,dc->bnkc', lc[j], Wl)
        M = jnp.max(gT + L, axis=2)
        p = jax.nn.relu(M + bs[i][None, None, :]
                        - jnp.einsum('bnd,dc->bnc', s[i + 1], Wg))

    cls_one = jnp.repeat(cls_label[:, None, :], N, axis=1)
    x = jnp.concatenate([p, cls_one], axis=-1).reshape(B * N, -1)
    out = _head(x, fc1_v, fc1_g, fc1_b, fc2_v, fc2_g, fc2_b,
                fc3_v, fc3_g, fc3_b).reshape(B, N, 40)
    return (out, jnp.asarray(0.0, jnp.float32), lc_cons)


# speedup vs baseline: 11.6058x; 10.9391x over previous
import functools
import jax
import jax.numpy as jnp
from jax import lax
from jax.experimental import pallas as pl
from jax.experimental.pallas import tpu as pltpu
from jax.experimental.pallas import tpu_sc as plsc

KNN = 17
NW = 32


@functools.partial(jax.jit, static_argnums=(2, 3))
def _sc_gather(table, idx, CH, nch):
    E = idx.shape[0]
    D = table.shape[1]
    mesh = plsc.VectorSubcoreMesh(core_axis_name="c", subcore_axis_name="s")

    @functools.partial(
        pl.kernel,
        mesh=mesh,
        compiler_params=pltpu.CompilerParams(use_tc_tiling_on_sc=False),
        out_type=jax.ShapeDtypeStruct((E, D), table.dtype),
        scratch_types=[
            pltpu.VMEM((CH,), jnp.int32),
            pltpu.VMEM((CH, D), table.dtype),
            pltpu.SemaphoreType.DMA,
        ],
    )
    def k(table_hbm, idx_hbm, out_hbm, idx_v, rows_v, gsem):
        wid = lax.axis_index("s") * 2 + lax.axis_index("c")
        base = wid * (CH * nch)

        def body(i, _):
            off = base + i * CH
            pltpu.sync_copy(idx_hbm.at[pl.ds(off, CH)], idx_v)
            pltpu.async_copy(table_hbm.at[idx_v], rows_v, gsem).wait()
            pltpu.sync_copy(rows_v, out_hbm.at[pl.ds(off, CH)])
            return ()

        lax.fori_loop(0, nch, body, (), unroll=False)

    return k(table, idx)


def _gather(table, idx):
    E = idx.shape[0]
    per_w = E // NW
    CH = 128 if per_w % 128 == 0 else per_w
    nch = per_w // CH
    return _sc_gather(table, idx, CH, nch)


def _head_body(x_ref, v1_ref, s1_ref, b1_ref, v2_ref, s2_ref, b2_ref,
               v3_ref, s3_ref, b3_ref, o_ref):
    x = x_ref[...]
    h = jnp.dot(x, v1_ref[...].T, preferred_element_type=jnp.float32)
    h = jnp.maximum(h * s1_ref[...] + b1_ref[...], 0.0)
    h = jnp.dot(h, v2_ref[...].T, preferred_element_type=jnp.float32)
    h = jnp.maximum(h * s2_ref[...] + b2_ref[...], 0.0)
    h = jnp.dot(h, v3_ref[...].T, preferred_element_type=jnp.float32)
    h = h * s3_ref[...] + b3_ref[...]
    m = jnp.max(h, axis=-1, keepdims=True)
    z = h - m
    lse = jnp.log(jnp.sum(jnp.exp(z), axis=-1, keepdims=True))
    o_ref[...] = z - lse


def _head(x, fc1_v, fc1_g, fc1_b, fc2_v, fc2_g, fc2_b, fc3_v, fc3_g, fc3_b):
    R, C = x.shape
    BR = 2048
    s1 = (fc1_g / jnp.linalg.norm(fc1_v, axis=1))[None, :]
    s2 = (fc2_g / jnp.linalg.norm(fc2_v, axis=1))[None, :]
    s3 = (fc3_g / jnp.linalg.norm(fc3_v, axis=1))[None, :]
    full = lambda shape: pl.BlockSpec(shape, lambda i: (0, 0))
    return pl.pallas_call(
        _head_body,
        grid=(R // BR,),
        in_specs=[
            pl.BlockSpec((BR, C), lambda i: (i, 0)),
            full(fc1_v.shape), full((1, 512)), full((1, 512)),
            full(fc2_v.shape), full((1, 256)), full((1, 256)),
            full(fc3_v.shape), full((1, 40)), full((1, 40)),
        ],
        out_specs=pl.BlockSpec((BR, 40), lambda i: (i, 0)),
        out_shape=jax.ShapeDtypeStruct((R, 40), jnp.float32),
    )(x, fc1_v, s1, fc1_b[None, :], fc2_v, s2, fc2_b[None, :],
      fc3_v, s3, fc3_b[None, :])


def kernel(xyz, neighbors, data_idxes, local_axises, cls_label, W0, b0, W02,
           b02, W1, b1, W12, b12, W2, b2, fc1_v, fc1_g, fc1_b, fc2_v, fc2_g,
           fc2_b, fc3_v, fc3_g, fc3_b):
    B, N, _ = xyz.shape
    K = KNN
    R = B * N
    boff = (jnp.arange(B, dtype=jnp.int32) * N)[:, None]

    nb = [(neighbors[:, j * N:(j + 1) * N, 0:K].astype(jnp.int32)
           + boff[:, :, None]).reshape(-1) for j in range(4)]
    di = [(data_idxes[:, j * N:(j + 1) * N].astype(jnp.int32)
           + boff).reshape(-1) for j in range(4)]
    A = [local_axises[:, j * N:(j + 1) * N].reshape(R, 3, 3) for j in range(4)]

    def pad16(t):
        return jnp.pad(t, ((0, 0), (0, 16 - t.shape[1])))

    xyzf = xyz.reshape(R, 3)
    xyzp = pad16(xyzf)

    c1 = _gather(xyzp, di[0])
    c2 = _gather(c1, di[1])
    c3 = _gather(c2, di[2])
    c4 = _gather(c3, di[3])
    c = [c1, c2, c3, c4]
    s1 = c1
    s2 = _gather(s1, di[0])
    s3 = _gather(s2, di[1])
    s4 = _gather(s3, di[2])
    s5 = _gather(s4, di[3])
    s = [xyzp, s1, s2, s3, s4, s5]

    packed = jnp.concatenate([
        jnp.concatenate([c[j][:, 0:3], A[j].reshape(R, 9)], axis=1)
        for j in range(4)], axis=0)
    packed = pad16(packed)
    nb_all = jnp.concatenate([nb[j] + j * R for j in range(4)], axis=0)
    ge = _gather(packed, nb_all).reshape(4, B, N, K, 16)

    lc_cons = jnp.asarray(0.0, jnp.float32)
    lcs = []
    for j in range(4):
        gx = ge[j, :, :, :, 0:3]
        cc = c[j][:, 0:3].reshape(B, N, 3)
        Aj = A[j].reshape(B, N, 3, 3)
        lcs.append(jnp.einsum('bnkd,bndc->bnkc', gx - cc[:, :, None, :], Aj))
        ga = ge[j, :, :, :, 3:12].reshape(B * N, K, 3, 3)
        t = jnp.cos(jnp.asarray(15.0 * (j + 1) * 3.141592653 / 180.0,
                                jnp.float32))
        for col in (0, 1):
            v = ga[:, :, :, col]
            g = jnp.matmul(v, jnp.transpose(v, (0, 2, 1)))
            m = g < t
            cnt = jnp.sum(m)
            sm = jnp.sum(jnp.where(m, g, 0.0))
            lc_cons = lc_cons + jnp.where(
                cnt > 0, sm / jnp.maximum(cnt, 1).astype(jnp.float32), 0.0)

    Ws = [W0, W02, W1, W12, W2]
    bs = [b0, b02, b1, b12, b2]
    lidx = [0, 0, 1, 2, 3]
    sdi = [di[0], di[0], di[1], di[2], di[3]]
    p = None
    for i in range(5):
        W = Ws[i]
        Wl, Wg = W[0:3], W[3:6]
        T = jnp.dot(s[i][:, 0:3], Wg, preferred_element_type=jnp.float32)
        if p is not None:
            T = T + jnp.dot(p, W[6:], preferred_element_type=jnp.float32)
        T_l = _gather(T, sdi[i])
        gT = _gather(T_l, nb[lidx[i]]).reshape(B, N, K, -1)
        L = jnp.einsum('bnkd,dc->bnkc', lcs[lidx[i]], Wl)
        M = jnp.max(gT + L, axis=2).reshape(R, -1)
        ctr = jnp.dot(s[i + 1][:, 0:3], Wg,
                      preferred_element_type=jnp.float32)
        p = jax.nn.relu(M + bs[i][None, :] - ctr)

    cls_one = jnp.repeat(cls_label[:, None, :], N, axis=1).reshape(R, -1)
    x = jnp.concatenate([p, cls_one], axis=-1)
    out = _head(x, fc1_v, fc1_g, fc1_b, fc2_v, fc2_g, fc2_b,
                fc3_v, fc3_g, fc3_b).reshape(B, N, 40)
    return (out, jnp.asarray(0.0, jnp.float32), lc_cons)


# speedup vs baseline: 12.6631x; 1.0911x over previous
import functools
import jax
import jax.numpy as jnp
from jax import lax
from jax.experimental import pallas as pl
from jax.experimental.pallas import tpu as pltpu
from jax.experimental.pallas import tpu_sc as plsc

KNN = 17
NW = 32


@functools.partial(jax.jit, static_argnums=(2, 3))
def _sc_gather(table, idx, CH, nch):
    E = idx.shape[0]
    D = table.shape[1]
    per_w = CH * nch
    mesh = plsc.VectorSubcoreMesh(core_axis_name="c", subcore_axis_name="s")

    @functools.partial(
        pl.kernel,
        mesh=mesh,
        compiler_params=pltpu.CompilerParams(use_tc_tiling_on_sc=False),
        out_type=jax.ShapeDtypeStruct((E, D), table.dtype),
        scratch_types=[
            pltpu.VMEM((per_w,), jnp.int32),
            pltpu.VMEM((2, CH, D), table.dtype),
            pltpu.SemaphoreType.DMA,
            pltpu.SemaphoreType.DMA,
            pltpu.SemaphoreType.DMA,
        ],
    )
    def k(table_hbm, idx_hbm, out_hbm, idx_v, rows_v, gsem, osem0, osem1):
        wid = lax.axis_index("s") * 2 + lax.axis_index("c")
        base = wid * per_w
        pltpu.sync_copy(idx_hbm.at[pl.ds(base, per_w)], idx_v)

        def chunk(i, slot, osem, first):
            @pl.when(jnp.logical_not(first))
            def _():
                pltpu.make_async_copy(
                    rows_v.at[slot],
                    out_hbm.at[pl.ds(base + (i - 2) * CH, CH)], osem).wait()
            pltpu.async_copy(
                table_hbm.at[idx_v.at[pl.ds(i * CH, CH)]],
                rows_v.at[slot], gsem).wait()
            pltpu.async_copy(rows_v.at[slot],
                             out_hbm.at[pl.ds(base + i * CH, CH)], osem)

        def body2(t, _):
            chunk(2 * t, 0, osem0, t == 0)
            @pl.when(2 * t + 1 < nch)
            def _():
                chunk(2 * t + 1, 1, osem1, t == 0)
            return ()

        lax.fori_loop(0, (nch + 1) // 2, body2, (), unroll=False)
        pltpu.make_async_copy(
            rows_v.at[(nch - 1) % 2],
            out_hbm.at[pl.ds(base + (nch - 1) * CH, CH)],
            osem1 if (nch - 1) % 2 == 1 else osem0).wait()
        if nch > 1:
            pltpu.make_async_copy(
                rows_v.at[(nch - 2) % 2],
                out_hbm.at[pl.ds(base + (nch - 2) * CH, CH)],
                osem1 if (nch - 2) % 2 == 1 else osem0).wait()

    return k(table, idx)


def _gather(table, idx):
    E = idx.shape[0]
    D = table.shape[1]
    per_w = E // NW
    cap = max(8, (200 * 1024) // (D * 4))
    CH = max(d for d in range(8, per_w + 1, 8)
             if per_w % d == 0 and d <= cap)
    nch = per_w // CH
    return _sc_gather(table, idx, CH, nch)


def _head_body(x_ref, v1_ref, s1_ref, b1_ref, v2_ref, s2_ref, b2_ref,
               v3_ref, s3_ref, b3_ref, o_ref):
    x = x_ref[...]
    h = jnp.dot(x, v1_ref[...].T, preferred_element_type=jnp.float32)
    h = jnp.maximum(h * s1_ref[...] + b1_ref[...], 0.0)
    h = jnp.dot(h, v2_ref[...].T, preferred_element_type=jnp.float32)
    h = jnp.maximum(h * s2_ref[...] + b2_ref[...], 0.0)
    h = jnp.dot(h, v3_ref[...].T, preferred_element_type=jnp.float32)
    h = h * s3_ref[...] + b3_ref[...]
    m = jnp.max(h, axis=-1, keepdims=True)
    z = h - m
    lse = jnp.log(jnp.sum(jnp.exp(z), axis=-1, keepdims=True))
    o_ref[...] = z - lse


def _head(x, fc1_v, fc1_g, fc1_b, fc2_v, fc2_g, fc2_b, fc3_v, fc3_g, fc3_b):
    R, C = x.shape
    BR = 2048
    s1 = (fc1_g / jnp.linalg.norm(fc1_v, axis=1))[None, :]
    s2 = (fc2_g / jnp.linalg.norm(fc2_v, axis=1))[None, :]
    s3 = (fc3_g / jnp.linalg.norm(fc3_v, axis=1))[None, :]
    full = lambda shape: pl.BlockSpec(shape, lambda i: (0, 0))
    return pl.pallas_call(
        _head_body,
        grid=(R // BR,),
        in_specs=[
            pl.BlockSpec((BR, C), lambda i: (i, 0)),
            full(fc1_v.shape), full((1, 512)), full((1, 512)),
            full(fc2_v.shape), full((1, 256)), full((1, 256)),
            full(fc3_v.shape), full((1, 40)), full((1, 40)),
        ],
        out_specs=pl.BlockSpec((BR, 40), lambda i: (i, 0)),
        out_shape=jax.ShapeDtypeStruct((R, 40), jnp.float32),
    )(x, fc1_v, s1, fc1_b[None, :], fc2_v, s2, fc2_b[None, :],
      fc3_v, s3, fc3_b[None, :])


def kernel(xyz, neighbors, data_idxes, local_axises, cls_label, W0, b0, W02,
           b02, W1, b1, W12, b12, W2, b2, fc1_v, fc1_g, fc1_b, fc2_v, fc2_g,
           fc2_b, fc3_v, fc3_g, fc3_b):
    B, N, _ = xyz.shape
    K = KNN
    R = B * N
    boff = (jnp.arange(B, dtype=jnp.int32) * N)[:, None]

    nb = [(neighbors[:, j * N:(j + 1) * N, 0:K].astype(jnp.int32)
           + boff[:, :, None]).reshape(-1) for j in range(4)]
    di = [(data_idxes[:, j * N:(j + 1) * N].astype(jnp.int32)
           + boff).reshape(-1) for j in range(4)]
    A = [local_axises[:, j * N:(j + 1) * N].reshape(R, 3, 3) for j in range(4)]

    def pad16(t):
        return jnp.pad(t, ((0, 0), (0, 16 - t.shape[1])))

    xyzf = xyz.reshape(R, 3)
    xyzp = pad16(xyzf)

    c1 = _gather(xyzp, di[0])
    c2 = _gather(c1, di[1])
    c3 = _gather(c2, di[2])
    c4 = _gather(c3, di[3])
    c = [c1, c2, c3, c4]
    s1 = c1
    s2 = _gather(s1, di[0])
    s3 = _gather(s2, di[1])
    s4 = _gather(s3, di[2])
    s5 = _gather(s4, di[3])
    s = [xyzp, s1, s2, s3, s4, s5]

    packed = jnp.concatenate([
        jnp.concatenate([c[j][:, 0:3], A[j].reshape(R, 9)], axis=1)
        for j in range(4)], axis=0)
    packed = pad16(packed)
    nb_all = jnp.concatenate([nb[j] + j * R for j in range(4)], axis=0)
    ge = _gather(packed, nb_all).reshape(4, B, N, K, 16)

    lc_cons = jnp.asarray(0.0, jnp.float32)
    lcs = []
    for j in range(4):
        gx = ge[j, :, :, :, 0:3]
        cc = c[j][:, 0:3].reshape(B, N, 3)
        Aj = A[j].reshape(B, N, 3, 3)
        lcs.append(jnp.einsum('bnkd,bndc->bnkc', gx - cc[:, :, None, :], Aj))
        ga = ge[j, :, :, :, 3:12].reshape(B * N, K, 3, 3)
        t = jnp.cos(jnp.asarray(15.0 * (j + 1) * 3.141592653 / 180.0,
                                jnp.float32))
        for col in (0, 1):
            v = ga[:, :, :, col]
            g = jnp.matmul(v, jnp.transpose(v, (0, 2, 1)))
            m = g < t
            cnt = jnp.sum(m)
            sm = jnp.sum(jnp.where(m, g, 0.0))
            lc_cons = lc_cons + jnp.where(
                cnt > 0, sm / jnp.maximum(cnt, 1).astype(jnp.float32), 0.0)

    Ws = [W0, W02, W1, W12, W2]
    bs = [b0, b02, b1, b12, b2]
    lidx = [0, 0, 1, 2, 3]
    sdi = [di[0], di[0], di[1], di[2], di[3]]
    p = None
    for i in range(5):
        W = Ws[i]
        Wl, Wg = W[0:3], W[3:6]
        T = jnp.dot(s[i][:, 0:3], Wg, preferred_element_type=jnp.float32)
        if p is not None:
            T = T + jnp.dot(p, W[6:], preferred_element_type=jnp.float32)
        T_l = _gather(T, sdi[i])
        gT = _gather(T_l, nb[lidx[i]]).reshape(B, N, K, -1)
        L = jnp.einsum('bnkd,dc->bnkc', lcs[lidx[i]], Wl)
        M = jnp.max(gT + L, axis=2).reshape(R, -1)
        ctr = jnp.dot(s[i + 1][:, 0:3], Wg,
                      preferred_element_type=jnp.float32)
        p = jax.nn.relu(M + bs[i][None, :] - ctr)

    cls_one = jnp.repeat(cls_label[:, None, :], N, axis=1).reshape(R, -1)
    x = jnp.concatenate([p, cls_one], axis=-1)
    out = _head(x, fc1_v, fc1_g, fc1_b, fc2_v, fc2_g, fc2_b,
                fc3_v, fc3_g, fc3_b).reshape(B, N, 40)
    return (out, jnp.asarray(0.0, jnp.float32), lc_cons)


# speedup vs baseline: 17.0710x; 1.3481x over previous
import functools
import jax
import jax.numpy as jnp
from jax import lax
from jax.experimental import pallas as pl
from jax.experimental.pallas import tpu as pltpu
from jax.experimental.pallas import tpu_sc as plsc

KNN = 17
NW = 32


@functools.partial(jax.jit, static_argnums=(2, 3))
def _sc_gather(table, idx, CH, nch):
    E = idx.shape[0]
    D = table.shape[1]
    per_w = CH * nch
    mesh = plsc.VectorSubcoreMesh(core_axis_name="c", subcore_axis_name="s")

    @functools.partial(
        pl.kernel,
        mesh=mesh,
        compiler_params=pltpu.CompilerParams(use_tc_tiling_on_sc=False),
        out_type=jax.ShapeDtypeStruct((E, D), table.dtype),
        scratch_types=[
            pltpu.VMEM((per_w,), jnp.int32),
            pltpu.VMEM((2, CH, D), table.dtype),
            pltpu.SemaphoreType.DMA,
            pltpu.SemaphoreType.DMA,
            pltpu.SemaphoreType.DMA,
        ],
    )
    def k(table_hbm, idx_hbm, out_hbm, idx_v, rows_v, gsem, osem0, osem1):
        wid = lax.axis_index("s") * 2 + lax.axis_index("c")
        base = wid * per_w
        pltpu.sync_copy(idx_hbm.at[pl.ds(base, per_w)], idx_v)

        def chunk(i, slot, osem, first):
            @pl.when(jnp.logical_not(first))
            def _():
                pltpu.make_async_copy(
                    rows_v.at[slot],
                    out_hbm.at[pl.ds(base + (i - 2) * CH, CH)], osem).wait()
            pltpu.async_copy(
                table_hbm.at[idx_v.at[pl.ds(i * CH, CH)]],
                rows_v.at[slot], gsem).wait()
            pltpu.async_copy(rows_v.at[slot],
                             out_hbm.at[pl.ds(base + i * CH, CH)], osem)

        def body2(t, _):
            chunk(2 * t, 0, osem0, t == 0)
            @pl.when(2 * t + 1 < nch)
            def _():
                chunk(2 * t + 1, 1, osem1, t == 0)
            return ()

        lax.fori_loop(0, (nch + 1) // 2, body2, (), unroll=False)
        pltpu.make_async_copy(
            rows_v.at[(nch - 1) % 2],
            out_hbm.at[pl.ds(base + (nch - 1) * CH, CH)],
            osem1 if (nch - 1) % 2 == 1 else osem0).wait()
        if nch > 1:
            pltpu.make_async_copy(
                rows_v.at[(nch - 2) % 2],
                out_hbm.at[pl.ds(base + (nch - 2) * CH, CH)],
                osem1 if (nch - 2) % 2 == 1 else osem0).wait()

    return k(table, idx)


def _gather(table, idx):
    E = idx.shape[0]
    D = table.shape[1]
    per_w = E // NW
    cap = max(8, (200 * 1024) // (D * 4))
    CH = max(d for d in range(8, per_w + 1, 8)
             if per_w % d == 0 and d <= cap)
    nch = per_w // CH
    return _sc_gather(table, idx, CH, nch)


PPC = 8
EPP = KNN


@functools.partial(jax.jit, static_argnums=(4,))
def _sc_layer(PT, nb, AC, Wl, C):
    R = PT.shape[0]
    D = C + 16
    per_pt = R // NW
    nch = per_pt // PPC
    CH = PPC * EPP
    mesh = plsc.VectorSubcoreMesh(core_axis_name="c", subcore_axis_name="s")

    @functools.partial(
        pl.kernel,
        mesh=mesh,
        compiler_params=pltpu.CompilerParams(use_tc_tiling_on_sc=False),
        out_type=jax.ShapeDtypeStruct((R, C), jnp.float32),
        scratch_types=[
            pltpu.VMEM((per_pt * EPP,), jnp.int32),
            pltpu.VMEM((per_pt, 16), jnp.float32),
            pltpu.VMEM((3 * C,), jnp.float32),
            pltpu.VMEM((2, CH, D), jnp.float32),
            pltpu.VMEM((2, PPC, C), jnp.float32),
            pltpu.SemaphoreType.DMA,
            pltpu.SemaphoreType.DMA,
            pltpu.SemaphoreType.DMA,
            pltpu.SemaphoreType.DMA,
        ],
    )
    def k(PT_h, nb_h, AC_h, Wl_h, M_h, idx_v, ac_v, wl_v, rows_v, m_v,
          gsem0, gsem1, osem0, osem1):
        wid = lax.axis_index("s") * 2 + lax.axis_index("c")
        pbase = wid * per_pt
        ebase = pbase * EPP
        pltpu.sync_copy(nb_h.at[pl.ds(ebase, per_pt * EPP)], idx_v)
        pltpu.sync_copy(AC_h.at[pl.ds(pbase, per_pt)], ac_v)
        pltpu.sync_copy(Wl_h, wl_v)

        def issue_gather(ch, slot):
            pltpu.async_copy(
                PT_h.at[idx_v.at[pl.ds(ch * CH, CH)]],
                rows_v.at[slot], gsem0 if slot == 0 else gsem1)

        def compute(ch, slot):
            rv = rows_v.at[slot]
            mv = m_v.at[slot]
            for p in range(PPC):
                pw = ch * PPC + p
                av = ac_v[pw, pl.ds(0, 16)]
                a = [av[t] for t in range(9)]
                ctr = [av[9 + d] for d in range(3)]

                def ebody(e, acc):
                    r = p * EPP + e
                    tail = rv[r, pl.ds(C, 16)]
                    dx = [tail[d] - ctr[d] for d in range(3)]
                    lc = [dx[0] * a[cc] + dx[1] * a[3 + cc]
                          + dx[2] * a[6 + cc] for cc in range(3)]
                    out = []
                    for c2 in range(C // 16):
                        v = rv[r, pl.ds(c2 * 16, 16)]
                        v = (v + lc[0] * wl_v[pl.ds(c2 * 16, 16)]
                             + lc[1] * wl_v[pl.ds(C + c2 * 16, 16)]
                             + lc[2] * wl_v[pl.ds(2 * C + c2 * 16, 16)])
                        out.append(jnp.maximum(acc[c2], v))
                    return tuple(out)

                acc = lax.fori_loop(
                    0, EPP, ebody,
                    tuple(jnp.full((16,), -3.4e38, jnp.float32)
                          for _ in range(C // 16)))
                for c2 in range(C // 16):
                    mv[p, pl.ds(c2 * 16, 16)] = acc[c2]

        def body2(t, _):
            for sl in (0, 1):
                ch = 2 * t + sl
                gsem = gsem0 if sl == 0 else gsem1
                osem = osem0 if sl == 0 else osem1
                pltpu.make_async_copy(
                    PT_h.at[idx_v.at[pl.ds(ch * CH, CH)]],
                    rows_v.at[sl], gsem).wait()
                @pl.when(t > 0)
                def _():
                    pltpu.make_async_copy(
                        m_v.at[sl],
                        M_h.at[pl.ds(pbase + (ch - 2) * PPC, PPC)],
                        osem).wait()
                compute(ch, sl)
                pltpu.async_copy(
                    m_v.at[sl], M_h.at[pl.ds(pbase + ch * PPC, PPC)], osem)
                @pl.when(ch + 2 < nch)
                def _():
                    issue_gather(ch + 2, sl)
            return ()

        issue_gather(0, 0)
        issue_gather(1, 1)
        lax.fori_loop(0, nch // 2, body2, (), unroll=False)
        pltpu.make_async_copy(
            m_v.at[0], M_h.at[pl.ds(pbase + (nch - 2) * PPC, PPC)],
            osem0).wait()
        pltpu.make_async_copy(
            m_v.at[1], M_h.at[pl.ds(pbase + (nch - 1) * PPC, PPC)],
            osem1).wait()

    return k(PT, nb, AC, Wl)


def _head_body(x_ref, v1_ref, s1_ref, b1_ref, v2_ref, s2_ref, b2_ref,
               v3_ref, s3_ref, b3_ref, o_ref):
    x = x_ref[...]
    h = jnp.dot(x, v1_ref[...].T, preferred_element_type=jnp.float32)
    h = jnp.maximum(h * s1_ref[...] + b1_ref[...], 0.0)
    h = jnp.dot(h, v2_ref[...].T, preferred_element_type=jnp.float32)
    h = jnp.maximum(h * s2_ref[...] + b2_ref[...], 0.0)
    h = jnp.dot(h, v3_ref[...].T, preferred_element_type=jnp.float32)
    h = h * s3_ref[...] + b3_ref[...]
    m = jnp.max(h, axis=-1, keepdims=True)
    z = h - m
    lse = jnp.log(jnp.sum(jnp.exp(z), axis=-1, keepdims=True))
    o_ref[...] = z - lse


def _head(x, fc1_v, fc1_g, fc1_b, fc2_v, fc2_g, fc2_b, fc3_v, fc3_g, fc3_b):
    R, C = x.shape
    BR = 2048
    s1 = (fc1_g / jnp.linalg.norm(fc1_v, axis=1))[None, :]
    s2 = (fc2_g / jnp.linalg.norm(fc2_v, axis=1))[None, :]
    s3 = (fc3_g / jnp.linalg.norm(fc3_v, axis=1))[None, :]
    full = lambda shape: pl.BlockSpec(shape, lambda i: (0, 0))
    return pl.pallas_call(
        _head_body,
        grid=(R // BR,),
        in_specs=[
            pl.BlockSpec((BR, C), lambda i: (i, 0)),
            full(fc1_v.shape), full((1, 512)), full((1, 512)),
            full(fc2_v.shape), full((1, 256)), full((1, 256)),
            full(fc3_v.shape), full((1, 40)), full((1, 40)),
        ],
        out_specs=pl.BlockSpec((BR, 40), lambda i: (i, 0)),
        out_shape=jax.ShapeDtypeStruct((R, 40), jnp.float32),
    )(x, fc1_v, s1, fc1_b[None, :], fc2_v, s2, fc2_b[None, :],
      fc3_v, s3, fc3_b[None, :])


def kernel(xyz, neighbors, data_idxes, local_axises, cls_label, W0, b0, W02,
           b02, W1, b1, W12, b12, W2, b2, fc1_v, fc1_g, fc1_b, fc2_v, fc2_g,
           fc2_b, fc3_v, fc3_g, fc3_b):
    B, N, _ = xyz.shape
    K = KNN
    R = B * N
    boff = (jnp.arange(B, dtype=jnp.int32) * N)[:, None]

    nb = [(neighbors[:, j * N:(j + 1) * N, 0:K].astype(jnp.int32)
           + boff[:, :, None]).reshape(-1) for j in range(4)]
    di = [(data_idxes[:, j * N:(j + 1) * N].astype(jnp.int32)
           + boff).reshape(-1) for j in range(4)]
    A = [local_axises[:, j * N:(j + 1) * N].reshape(R, 3, 3) for j in range(4)]

    def pad16(t):
        return jnp.pad(t, ((0, 0), (0, 16 - t.shape[1])))

    xyzf = xyz.reshape(R, 3)
    xyzp = pad16(xyzf)

    c1 = _gather(xyzp, di[0])
    c2 = _gather(c1, di[1])
    c3 = _gather(c2, di[2])
    c4 = _gather(c3, di[3])
    c = [c1, c2, c3, c4]
    s1 = c1
    s2 = _gather(s1, di[0])
    s3 = _gather(s2, di[1])
    s4 = _gather(s3, di[2])
    s5 = _gather(s4, di[3])
    s = [xyzp, s1, s2, s3, s4, s5]

    packed = jnp.concatenate([
        jnp.concatenate([c[j][:, 0:3], A[j].reshape(R, 9)], axis=1)
        for j in range(4)], axis=0)
    packed = pad16(packed)
    nb_all = jnp.concatenate([nb[j] + j * R for j in range(4)], axis=0)
    ge = _gather(packed, nb_all).reshape(4, B, N, K, 16)

    lc_cons = jnp.asarray(0.0, jnp.float32)
    for j in range(4):
        ga = ge[j, :, :, :, 3:12].reshape(B * N, K, 3, 3)
        t = jnp.cos(jnp.asarray(15.0 * (j + 1) * 3.141592653 / 180.0,
                                jnp.float32))
        for col in (0, 1):
            v = ga[:, :, :, col]
            g = jnp.matmul(v, jnp.transpose(v, (0, 2, 1)))
            m = g < t
            cnt = jnp.sum(m)
            sm = jnp.sum(jnp.where(m, g, 0.0))
            lc_cons = lc_cons + jnp.where(
                cnt > 0, sm / jnp.maximum(cnt, 1).astype(jnp.float32), 0.0)

    Ws = [W0, W02, W1, W12, W2]
    bs = [b0, b02, b1, b12, b2]
    lidx = [0, 0, 1, 2, 3]
    sdi = [di[0], di[0], di[1], di[2], di[3]]
    AC = [jnp.concatenate(
        [A[j].reshape(R, 9), c[j][:, 0:3], jnp.zeros((R, 4), jnp.float32)],
        axis=1) for j in range(4)]
    p = None
    for i in range(5):
        W = Ws[i]
        Wl, Wg = W[0:3], W[3:6]
        C = W.shape[1]
        T = jnp.dot(s[i][:, 0:3], Wg, preferred_element_type=jnp.float32)
        if p is not None:
            T = T + jnp.dot(p, W[6:], preferred_element_type=jnp.float32)
        T_l = _gather(T, sdi[i])
        j = lidx[i]
        PT = jnp.concatenate([T_l[:, 0:C], c[j][:, 0:16]], axis=1)
        M = _sc_layer(PT, nb[j], AC[j], Wl.reshape(-1), C)
        ctr = jnp.dot(s[i + 1][:, 0:3], Wg,
                      preferred_element_type=jnp.float32)
        p = jax.nn.relu(M + bs[i][None, :] - ctr)

    cls_one = jnp.repeat(cls_label[:, None, :], N, axis=1).reshape(R, -1)
    x = jnp.concatenate([p, cls_one], axis=-1)
    out = _head(x, fc1_v, fc1_g, fc1_b, fc2_v, fc2_g, fc2_b,
                fc3_v, fc3_g, fc3_b).reshape(B, N, 40)
    return (out, jnp.asarray(0.0, jnp.float32), lc_cons)
